# baseline (speedup 1.0000x reference)
import jax
import jax.numpy as jnp
from jax.experimental import pallas as pl
from jax.experimental.pallas import tpu as pltpu


_ROWS = 128
_COLS = 32768
_ZROWS = 16


_HEAD = 8


def _decomp_dma(x_ref, season_ref, trend_ref, vbuf, tbuf, zbuf, sems):
    zbuf[...] = jnp.zeros_like(zbuf)
    copies = [pltpu.make_async_copy(
        x_ref.at[pl.ds(_HEAD, _ROWS - _HEAD)],
        season_ref.at[pl.ds(_HEAD, _ROWS - _HEAD)], sems.at[0])]
    sem_idx = 1
    row = _HEAD
    while row < _ROWS:
        n = min(_ZROWS, _ROWS - row)
        copies.append(pltpu.make_async_copy(
            zbuf.at[pl.ds(0, n)], trend_ref.at[pl.ds(row, n)],
            sems.at[sem_idx]))
        sem_idx += 1
        row += n
    head_in = pltpu.make_async_copy(
        x_ref.at[pl.ds(0, _HEAD)], vbuf, sems.at[sem_idx])
    head_in.start()
    for c in copies:
        c.start()
    head_in.wait()
    xa = vbuf[...]
    is_row0 = jax.lax.broadcasted_iota(jnp.int32, xa.shape, 0) == 0
    zero = jnp.zeros_like(xa)
    tbuf[...] = jnp.where(is_row0, xa, zero)
    vbuf[...] = jnp.where(is_row0, zero, xa)
    head_season = pltpu.make_async_copy(
        vbuf, season_ref.at[pl.ds(0, _HEAD)], sems.at[sem_idx + 1])
    head_trend = pltpu.make_async_copy(
        tbuf, trend_ref.at[pl.ds(0, _HEAD)], sems.at[sem_idx + 2])
    head_season.start()
    head_trend.start()
    for c in copies:
        c.wait()
    head_season.wait()
    head_trend.wait()


def kernel(x):
    season, trend = pl.pallas_call(
        _decomp_dma,
        in_specs=[pl.BlockSpec(memory_space=pl.ANY)],
        out_specs=[
            pl.BlockSpec(memory_space=pl.ANY),
            pl.BlockSpec(memory_space=pl.ANY),
        ],
        out_shape=[
            jax.ShapeDtypeStruct((_ROWS, _COLS), x.dtype),
            jax.ShapeDtypeStruct((_ROWS, _COLS), x.dtype),
        ],
        scratch_shapes=[
            pltpu.VMEM((_HEAD, _COLS), jnp.float32),
            pltpu.VMEM((_HEAD, _COLS), jnp.float32),
            pltpu.VMEM((_ZROWS, _COLS), jnp.float32),
            pltpu.SemaphoreType.DMA((16,)),
        ],
    )(x)
    return (season, trend)


# speedup vs baseline: 28.9061x; 28.9061x over previous
import jax
import jax.numpy as jnp
from jax.experimental import pallas as pl
from jax.experimental.pallas import tpu as pltpu


_ROWS = 128
_COLS = 32768
_BLOCK_ROWS = 32


def _decomp_block(x_ref, season_ref, trend_ref):
    x = x_ref[...]
    row = jax.lax.broadcasted_iota(jnp.int32, x.shape, 0)
    is_row0 = (row + pl.program_id(0) * _BLOCK_ROWS) == 0
    zero = jnp.zeros_like(x)
    season_ref[...] = jnp.where(is_row0, zero, x)
    trend_ref[...] = jnp.where(is_row0, x, zero)


def kernel(x):
    grid = (_ROWS // _BLOCK_ROWS,)
    spec = pl.BlockSpec((_BLOCK_ROWS, _COLS), lambda i: (i, 0))
    season, trend = pl.pallas_call(
        _decomp_block,
        grid=grid,
        in_specs=[spec],
        out_specs=[spec, spec],
        out_shape=[
            jax.ShapeDtypeStruct((_ROWS, _COLS), x.dtype),
            jax.ShapeDtypeStruct((_ROWS, _COLS), x.dtype),
        ],
        compiler_params=pltpu.CompilerParams(
            dimension_semantics=("parallel",),
        ),
    )(x)
    return (season, trend)
